# baseline (speedup 1.0000x reference)
import jax
import jax.numpy as jnp
from jax.experimental import pallas as pl
from jax.experimental.pallas import tpu as pltpu

_NCHUNK = 8


def _dma_copy(src_ref, dst_ref, sems):
    rows = src_ref.shape[0] // _NCHUNK
    for k in range(_NCHUNK):
        pltpu.make_async_copy(
            src_ref.at[pl.ds(k * rows, rows)],
            dst_ref.at[pl.ds(k * rows, rows)],
            sems.at[k],
        ).start()
    for k in range(_NCHUNK):
        pltpu.make_async_copy(
            src_ref.at[pl.ds(k * rows, rows)],
            dst_ref.at[pl.ds(k * rows, rows)],
            sems.at[k],
        ).wait()


def kernel(ref_bxyz, ref_feat, group_ids):
    del ref_bxyz, group_ids
    n, d = ref_feat.shape
    return pl.pallas_call(
        _dma_copy,
        in_specs=[pl.BlockSpec(memory_space=pl.ANY)],
        out_specs=pl.BlockSpec(memory_space=pl.ANY),
        scratch_shapes=[pltpu.SemaphoreType.DMA((_NCHUNK,))],
        out_shape=jax.ShapeDtypeStruct((n, d), ref_feat.dtype),
    )(ref_feat)


# speedup vs baseline: 47.5044x; 47.5044x over previous
import jax
import jax.numpy as jnp
from jax.experimental import pallas as pl

_BLK = 6400


def _copy_block(feat_ref, out_ref):
    out_ref[...] = feat_ref[...]


def kernel(ref_bxyz, ref_feat, group_ids):
    del ref_bxyz, group_ids
    n, d = ref_feat.shape
    grid = n // _BLK
    return pl.pallas_call(
        _copy_block,
        grid=(grid,),
        in_specs=[pl.BlockSpec((_BLK, d), lambda i: (i, 0))],
        out_specs=pl.BlockSpec((_BLK, d), lambda i: (i, 0)),
        out_shape=jax.ShapeDtypeStruct((n, d), ref_feat.dtype),
    )(ref_feat)


# speedup vs baseline: 48.9698x; 1.0308x over previous
import jax
import jax.numpy as jnp
from jax.experimental import pallas as pl

_BLK = 12800


def _copy_block(feat_ref, out_ref):
    out_ref[...] = feat_ref[...]


def kernel(ref_bxyz, ref_feat, group_ids):
    del ref_bxyz, group_ids
    n, d = ref_feat.shape
    grid = n // _BLK
    return pl.pallas_call(
        _copy_block,
        grid=(grid,),
        in_specs=[pl.BlockSpec((_BLK, d), lambda i: (i, 0))],
        out_specs=pl.BlockSpec((_BLK, d), lambda i: (i, 0)),
        out_shape=jax.ShapeDtypeStruct((n, d), ref_feat.dtype),
    )(ref_feat)


# speedup vs baseline: 49.0426x; 1.0015x over previous
import jax
import jax.numpy as jnp
from jax.experimental import pallas as pl

_BLK = 16000


def _copy_block(feat_ref, out_ref):
    out_ref[...] = feat_ref[...]


def kernel(ref_bxyz, ref_feat, group_ids):
    del ref_bxyz, group_ids
    n, d = ref_feat.shape
    grid = n // _BLK
    return pl.pallas_call(
        _copy_block,
        grid=(grid,),
        in_specs=[pl.BlockSpec((_BLK, d), lambda i: (i, 0))],
        out_specs=pl.BlockSpec((_BLK, d), lambda i: (i, 0)),
        out_shape=jax.ShapeDtypeStruct((n, d), ref_feat.dtype),
    )(ref_feat)


# speedup vs baseline: 49.1043x; 1.0013x over previous
import jax
import jax.numpy as jnp
from jax.experimental import pallas as pl

_BLK = 20000


def _copy_block(feat_ref, out_ref):
    out_ref[...] = feat_ref[...]


def kernel(ref_bxyz, ref_feat, group_ids):
    del ref_bxyz, group_ids
    n, d = ref_feat.shape
    grid = n // _BLK
    return pl.pallas_call(
        _copy_block,
        grid=(grid,),
        in_specs=[pl.BlockSpec((_BLK, d), lambda i: (i, 0))],
        out_specs=pl.BlockSpec((_BLK, d), lambda i: (i, 0)),
        out_shape=jax.ShapeDtypeStruct((n, d), ref_feat.dtype),
    )(ref_feat)
